# baseline (speedup 1.0000x reference)
import functools

import jax
import jax.numpy as jnp
from jax import lax
from jax.experimental import pallas as pl
from jax.experimental.pallas import tpu as pltpu
from jax.experimental.pallas import tpu_sc as plsc

N = 50000
E = 1600000
D = 128
HID = 1024
C = 10
ZC = 16

NC = 2
NS = 16
NW = NC * NS

N_PAD = 50048
JUNK_ROW = N
RPT = N_PAD // NS

B = 512
K = 4
NCH = 100
EPT = NCH * B
E_PAD = EPT * NW
ER = E_PAD // 128
LC = NCH - 1

MLP_R = 3128
EPI_R = 2000


def _mlp_body(x_ref, w1_ref, b1_ref, w2_ref, b2_ref, o_ref):
    x = x_ref[...]
    h = jnp.maximum(
        jnp.dot(x, w1_ref[...], preferred_element_type=jnp.float32)
        + b1_ref[...], 0.0)
    z = jnp.dot(h, w2_ref[...], preferred_element_type=jnp.float32) + b2_ref[...]
    ym = (x[:, 2:3] - x[:, 126:127]) * 10.0
    msk = ((x[:, 0:1] <= x[:, 5:6]) & (x[:, 1:2] >= x[:, 4:5])
           & (x[:, 2:3] <= x[:, 127:128]) & (x[:, 3:4] >= x[:, 126:127]))
    it = jnp.where(msk, 5.0, -5.0)
    col = lax.broadcasted_iota(jnp.int32, (MLP_R, ZC), 1)
    z = jnp.where(col == 8, ym, z)
    z = jnp.where(col == 9, it, z)
    o_ref[...] = z


_mlp = pl.pallas_call(
    _mlp_body,
    grid=(N_PAD // MLP_R,),
    in_specs=[
        pl.BlockSpec((MLP_R, D), lambda i: (i, 0)),
        pl.BlockSpec((D, HID), lambda i: (0, 0)),
        pl.BlockSpec((1, HID), lambda i: (0, 0)),
        pl.BlockSpec((HID, ZC), lambda i: (0, 0)),
        pl.BlockSpec((1, ZC), lambda i: (0, 0)),
    ],
    out_specs=pl.BlockSpec((MLP_R, ZC), lambda i: (i, 0)),
    out_shape=jax.ShapeDtypeStruct((N_PAD, ZC), jnp.float32),
)


def _comb_body(a_ref, b_ref, c_ref, o_ref):
    o_ref[...] = a_ref[...] + b_ref[...] + c_ref[...]


_combine = pl.pallas_call(
    _comb_body,
    grid=(N_PAD // MLP_R,),
    in_specs=[pl.BlockSpec((MLP_R, ZC), lambda i: (i, 0))] * 3,
    out_specs=pl.BlockSpec((MLP_R, ZC), lambda i: (i, 0)),
    out_shape=jax.ShapeDtypeStruct((N_PAD, ZC), jnp.float32),
)


def _epi_body(a_ref, b_ref, c_ref, o_ref):
    z = a_ref[...] + b_ref[...] + c_ref[...]
    za = z[:, 0:4]
    zb = z[:, 4:8]
    ea = jnp.exp(za - jnp.max(za, axis=1, keepdims=True))
    sa = ea / jnp.sum(ea, axis=1, keepdims=True)
    eb = jnp.exp(zb - jnp.max(zb, axis=1, keepdims=True))
    sb = eb / jnp.sum(eb, axis=1, keepdims=True)
    o_ref[...] = jnp.concatenate([za, sa, zb, sb], axis=1)


_epi = pl.pallas_call(
    _epi_body,
    grid=(N // EPI_R,),
    in_specs=[pl.BlockSpec((EPI_R, ZC), lambda i: (i, 0))] * 3,
    out_specs=pl.BlockSpec((EPI_R, ZC), lambda i: (i, 0)),
    out_shape=jax.ShapeDtypeStruct((N, ZC), jnp.float32),
)


def _kenn_body(zt, sx2, sy2, rel, ztab, wspb,
               out0, out1, relout,
               acc,
               sxq0, sxq1, syq0, syq1, rlq0, rlq1,
               zx0, zx1, zy0, zy1,
               dzx0, dzx1, dzy0, dzy1,
               rlo, wsp_v,
               semg0, semg1, sems0, sems1):
    cid = lax.axis_index("c")
    sid = lax.axis_index("s")
    wid = sid * NC + cid

    sxq = [sxq0, sxq1]
    syq = [syq0, syq1]
    rlq = [rlq0, rlq1]
    zx = [zx0, zx1]
    zy = [zy0, zy1]
    dzx = [dzx0, dzx1]
    dzy = [dzy0, dzy1]
    semg = [semg0, semg1]
    sems = [sems0, sems1]

    r0 = sid * RPT
    pltpu.sync_copy(ztab.at[pl.ds(r0, RPT)], acc.at[pl.ds(r0, RPT)])

    pltpu.sync_copy(wspb, wsp_v)
    wp = [wsp_v[pl.ds(c * 16, 16)] for c in range(C)]
    wn = [-w for w in wp]
    colv = [jnp.full((16,), c, jnp.int32) for c in range(ZC)]
    iota16 = lax.iota(jnp.int32, 16)
    zero16 = jnp.zeros((16,), jnp.float32)

    def zero_body(g, carry):
        rows = g * 16 + iota16
        for c in range(C, ZC):
            for buf in (dzx0, dzx1, dzy0, dzy1):
                plsc.store_scatter(buf, [rows, colv[c]], zero16)
        return carry

    lax.fori_loop(0, B // 16, zero_body, 0)
    plsc.subcore_barrier()

    def issue_idx(c, q):
        brow = wid * (EPT // 128) + c * K
        base = wid * EPT + c * B
        pltpu.sync_copy(sx2.at[pl.ds(brow, K)], sxq[q])
        pltpu.sync_copy(sy2.at[pl.ds(brow, K)], syq[q])
        pltpu.sync_copy(rel.at[pl.ds(base, B)], rlq[q])

    def issue_gather(q):
        for j in range(K):
            pltpu.async_copy(
                zt.at[sxq[q].at[j]], zx[q].at[pl.ds(j * 128, 128)], semg[q])
            pltpu.async_copy(
                zt.at[syq[q].at[j]], zy[q].at[pl.ds(j * 128, 128)], semg[q])

    def wait_gather(q):
        for j in range(K):
            pltpu.make_async_copy(
                zt.at[sxq[q].at[j]], zx[q].at[pl.ds(j * 128, 128)],
                semg[q]).wait()
            pltpu.make_async_copy(
                zt.at[syq[q].at[j]], zy[q].at[pl.ds(j * 128, 128)],
                semg[q]).wait()

    def issue_scatter(q):
        for j in range(K):
            pltpu.async_copy(dzx[q].at[pl.ds(j * 128, 128)],
                             acc.at[sxq[q].at[j]], sems[q], add=True)
            pltpu.async_copy(dzy[q].at[pl.ds(j * 128, 128)],
                             acc.at[syq[q].at[j]], sems[q], add=True)

    def wait_scatter(q):
        for j in range(K):
            pltpu.make_async_copy(
                dzx[q].at[pl.ds(j * 128, 128)], acc.at[sxq[q].at[j]],
                sems[q]).wait()
            pltpu.make_async_copy(
                dzy[q].at[pl.ds(j * 128, 128)], acc.at[syq[q].at[j]],
                sems[q]).wait()

    def compute(q):
        zxb, zyb, dzxb, dzyb, rlqb = zx[q], zy[q], dzx[q], dzy[q], rlq[q]

        def group_body(g, gcarry):
            rows = g * 16 + iota16
            relg = rlqb[pl.ds(g * 16, 16)]
            drel = jnp.zeros((16,), jnp.float32)
            for c in range(C):
                zxv = plsc.load_gather(zxb, [rows, colv[c]])
                zyv = plsc.load_gather(zyb, [rows, colv[c]])
                f0 = jnp.exp(jnp.minimum(-zxv - relg, 80.0))
                f2 = jnp.exp(jnp.minimum(zyv - relg, 80.0))
                r = 1.0 / (f0 + (1.0 + f2))
                plsc.store_scatter(dzxb, [rows, colv[c]], wn[c] * (f0 * r))
                plsc.store_scatter(dzyb, [rows, colv[c]], wp[c] * (f2 * r))
                drel = drel + wp[c] * r
            rlo[pl.ds(g * 16, 16)] = relg + drel * 0.1
            return gcarry

        lax.fori_loop(0, B // 16, group_body, 0)

    issue_idx(0, 0)
    issue_gather(0)

    def outer_body(c2, carry):
        for kk in range(2):
            c = c2 * 2 + kk
            wait_gather(kk)

            @pl.when(c >= 1)
            def _():
                wait_scatter(1 - kk)

            @pl.when(c + 1 <= LC)
            def _():
                issue_idx(c + 1, 1 - kk)
                issue_gather(1 - kk)

            compute(kk)
            base = wid * EPT + c * B
            pltpu.sync_copy(rlo, relout.at[pl.ds(base, B)])
            issue_scatter(kk)

        return carry

    lax.fori_loop(0, NCH // 2, outer_body, 0)
    wait_scatter(1)
    plsc.subcore_barrier()

    @pl.when(cid == 0)
    def _():
        pltpu.sync_copy(acc.at[pl.ds(r0, RPT)], out0.at[pl.ds(r0, RPT)])

    @pl.when(cid == 1)
    def _():
        pltpu.sync_copy(acc.at[pl.ds(r0, RPT)], out1.at[pl.ds(r0, RPT)])


_kenn_sc = functools.partial(
    pl.kernel,
    out_type=[
        jax.ShapeDtypeStruct((N_PAD, ZC), jnp.float32),
        jax.ShapeDtypeStruct((N_PAD, ZC), jnp.float32),
        jax.ShapeDtypeStruct((E_PAD,), jnp.float32),
    ],
    mesh=plsc.VectorSubcoreMesh(core_axis_name="c", subcore_axis_name="s"),
    compiler_params=pltpu.CompilerParams(
        needs_layout_passes=False, use_tc_tiling_on_sc=False),
    scratch_types=(
        [pltpu.VMEM_SHARED((N_PAD, ZC), jnp.float32)]
        + [pltpu.VMEM((K, 128), jnp.int32)] * 4
        + [pltpu.VMEM((B,), jnp.float32)] * 2
        + [pltpu.VMEM((B, ZC), jnp.float32)] * 8
        + [pltpu.VMEM((B,), jnp.float32)]
        + [pltpu.VMEM((C * 16,), jnp.float32)]
        + [pltpu.SemaphoreType.DMA] * 4
    ),
)(_kenn_body)


def kernel(features, relations, sx, sy, W1, b1, W2, b2, clause_w):
    f32 = jnp.float32
    fpad = jnp.concatenate(
        [features, jnp.zeros((N_PAD - N, D), f32)], axis=0)
    w2p = jnp.concatenate([W2, jnp.zeros((HID, ZC - 8), f32)], axis=1)
    b2p = jnp.concatenate([b2, jnp.zeros((ZC - 8,), f32)]).reshape(1, ZC)
    zt = _mlp(fpad, W1, b1.reshape(1, HID), w2p, b2p)

    pad_e = E_PAD - E
    sxp = jnp.concatenate(
        [sx, jnp.full((pad_e,), JUNK_ROW, jnp.int32)]).reshape(ER, 128)
    syp = jnp.concatenate(
        [sy, jnp.full((pad_e,), JUNK_ROW, jnp.int32)]).reshape(ER, 128)
    relc = jnp.concatenate([relations.reshape(-1), jnp.zeros((pad_e,), f32)])
    wspb = jnp.broadcast_to(
        jax.nn.softplus(clause_w)[:, :, None], (3, C, 16)).reshape(3, C * 16)
    ztab = jnp.zeros((N_PAD, ZC), f32)

    out0 = out1 = None
    for l in range(3):
        out0, out1, relc = _kenn_sc(zt, sxp, syp, relc, ztab, wspb[l])
        if l < 2:
            zt = _combine(zt, out0, out1)

    packed = _epi(zt, out0, out1)
    return (packed[:, 0:4], packed[:, 4:8], packed[:, 8:12], packed[:, 12:16])


# speedup vs baseline: 1.0449x; 1.0449x over previous
import functools

import jax
import jax.numpy as jnp
from jax import lax
from jax.experimental import pallas as pl
from jax.experimental.pallas import tpu as pltpu
from jax.experimental.pallas import tpu_sc as plsc

N = 50000
E = 1600000
D = 128
HID = 1024
C = 10
ZC = 16

NC = 2
NS = 16
NW = NC * NS

N_PAD = 50048
JUNK_ROW = N
RPT = N_PAD // NS

B = 768
K = 6
NCH = 66
EPT = NCH * B
E_PAD = EPT * NW
ER = E_PAD // 128
LC = NCH - 1

MLP_R = 3128
EPI_R = 2000


def _mlp_body(x_ref, w1_ref, b1_ref, w2_ref, b2_ref, o_ref):
    x = x_ref[...]
    h = jnp.maximum(
        jnp.dot(x, w1_ref[...], preferred_element_type=jnp.float32)
        + b1_ref[...], 0.0)
    z = jnp.dot(h, w2_ref[...], preferred_element_type=jnp.float32) + b2_ref[...]
    ym = (x[:, 2:3] - x[:, 126:127]) * 10.0
    msk = ((x[:, 0:1] <= x[:, 5:6]) & (x[:, 1:2] >= x[:, 4:5])
           & (x[:, 2:3] <= x[:, 127:128]) & (x[:, 3:4] >= x[:, 126:127]))
    it = jnp.where(msk, 5.0, -5.0)
    col = lax.broadcasted_iota(jnp.int32, (MLP_R, ZC), 1)
    z = jnp.where(col == 8, ym, z)
    z = jnp.where(col == 9, it, z)
    o_ref[...] = z


_mlp = pl.pallas_call(
    _mlp_body,
    grid=(N_PAD // MLP_R,),
    in_specs=[
        pl.BlockSpec((MLP_R, D), lambda i: (i, 0)),
        pl.BlockSpec((D, HID), lambda i: (0, 0)),
        pl.BlockSpec((1, HID), lambda i: (0, 0)),
        pl.BlockSpec((HID, ZC), lambda i: (0, 0)),
        pl.BlockSpec((1, ZC), lambda i: (0, 0)),
    ],
    out_specs=pl.BlockSpec((MLP_R, ZC), lambda i: (i, 0)),
    out_shape=jax.ShapeDtypeStruct((N_PAD, ZC), jnp.float32),
)


def _comb_body(a_ref, b_ref, c_ref, o_ref):
    o_ref[...] = a_ref[...] + b_ref[...] + c_ref[...]


_combine = pl.pallas_call(
    _comb_body,
    grid=(N_PAD // MLP_R,),
    in_specs=[pl.BlockSpec((MLP_R, ZC), lambda i: (i, 0))] * 3,
    out_specs=pl.BlockSpec((MLP_R, ZC), lambda i: (i, 0)),
    out_shape=jax.ShapeDtypeStruct((N_PAD, ZC), jnp.float32),
)


def _epi_body(a_ref, b_ref, c_ref, o_ref):
    z = a_ref[...] + b_ref[...] + c_ref[...]
    za = z[:, 0:4]
    zb = z[:, 4:8]
    ea = jnp.exp(za - jnp.max(za, axis=1, keepdims=True))
    sa = ea / jnp.sum(ea, axis=1, keepdims=True)
    eb = jnp.exp(zb - jnp.max(zb, axis=1, keepdims=True))
    sb = eb / jnp.sum(eb, axis=1, keepdims=True)
    o_ref[...] = jnp.concatenate([za, sa, zb, sb], axis=1)


_epi = pl.pallas_call(
    _epi_body,
    grid=(N // EPI_R,),
    in_specs=[pl.BlockSpec((EPI_R, ZC), lambda i: (i, 0))] * 3,
    out_specs=pl.BlockSpec((EPI_R, ZC), lambda i: (i, 0)),
    out_shape=jax.ShapeDtypeStruct((N, ZC), jnp.float32),
)


def _kenn_body(zt, sx2, sy2, rel, ztab, wspb,
               out0, out1, relout,
               acc,
               sxq0, sxq1, syq0, syq1, rlq0, rlq1,
               zx0, zx1, zy0, zy1,
               dzxb, dzyb,
               rlo, wsp_v,
               semg0, semg1, sems0):
    cid = lax.axis_index("c")
    sid = lax.axis_index("s")
    wid = sid * NC + cid

    sxq = [sxq0, sxq1]
    syq = [syq0, syq1]
    rlq = [rlq0, rlq1]
    zx = [zx0, zx1]
    zy = [zy0, zy1]
    semg = [semg0, semg1]

    r0 = sid * RPT
    pltpu.sync_copy(ztab.at[pl.ds(r0, RPT)], acc.at[pl.ds(r0, RPT)])

    pltpu.sync_copy(wspb, wsp_v)
    wp = [wsp_v[pl.ds(c * 16, 16)] for c in range(C)]
    wn = [-w for w in wp]
    colv = [jnp.full((16,), c, jnp.int32) for c in range(ZC)]
    iota16 = lax.iota(jnp.int32, 16)
    zero16 = jnp.zeros((16,), jnp.float32)

    def zero_body(g, carry):
        rows = g * 16 + iota16
        for c in range(C, ZC):
            for buf in (dzxb, dzyb):
                plsc.store_scatter(buf, [rows, colv[c]], zero16)
        return carry

    lax.fori_loop(0, B // 16, zero_body, 0)
    plsc.subcore_barrier()

    def issue_idx(c, q):
        brow = wid * (EPT // 128) + c * K
        base = wid * EPT + c * B
        pltpu.sync_copy(sx2.at[pl.ds(brow, K)], sxq[q])
        pltpu.sync_copy(sy2.at[pl.ds(brow, K)], syq[q])
        pltpu.sync_copy(rel.at[pl.ds(base, B)], rlq[q])

    def issue_gather(q):
        for j in range(K):
            pltpu.async_copy(
                zt.at[sxq[q].at[j]], zx[q].at[pl.ds(j * 128, 128)], semg[q])
            pltpu.async_copy(
                zt.at[syq[q].at[j]], zy[q].at[pl.ds(j * 128, 128)], semg[q])

    def wait_gather(q):
        for j in range(K):
            pltpu.make_async_copy(
                zt.at[sxq[q].at[j]], zx[q].at[pl.ds(j * 128, 128)],
                semg[q]).wait()
            pltpu.make_async_copy(
                zt.at[syq[q].at[j]], zy[q].at[pl.ds(j * 128, 128)],
                semg[q]).wait()

    def scatter_sync(q):
        for j in range(K):
            pltpu.async_copy(dzxb.at[pl.ds(j * 128, 128)],
                             acc.at[sxq[q].at[j]], sems0, add=True)
            pltpu.async_copy(dzyb.at[pl.ds(j * 128, 128)],
                             acc.at[syq[q].at[j]], sems0, add=True)
        for j in range(K):
            pltpu.make_async_copy(
                dzxb.at[pl.ds(j * 128, 128)], acc.at[sxq[q].at[j]],
                sems0).wait()
            pltpu.make_async_copy(
                dzyb.at[pl.ds(j * 128, 128)], acc.at[syq[q].at[j]],
                sems0).wait()

    def compute(q):
        zxb, zyb, rlqb = zx[q], zy[q], rlq[q]

        def group_body(g, gcarry):
            rows = g * 16 + iota16
            relg = rlqb[pl.ds(g * 16, 16)]
            drel = jnp.zeros((16,), jnp.float32)
            for c in range(C):
                zxv = plsc.load_gather(zxb, [rows, colv[c]])
                zyv = plsc.load_gather(zyb, [rows, colv[c]])
                f0 = jnp.exp(jnp.minimum(-zxv - relg, 80.0))
                f2 = jnp.exp(jnp.minimum(zyv - relg, 80.0))
                r = 1.0 / (f0 + (1.0 + f2))
                plsc.store_scatter(dzxb, [rows, colv[c]], wn[c] * (f0 * r))
                plsc.store_scatter(dzyb, [rows, colv[c]], wp[c] * (f2 * r))
                drel = drel + wp[c] * r
            rlo[pl.ds(g * 16, 16)] = relg + drel * 0.1
            return gcarry

        lax.fori_loop(0, B // 16, group_body, 0)

    issue_idx(0, 0)
    issue_gather(0)

    def outer_body(c2, carry):
        for kk in range(2):
            c = c2 * 2 + kk
            wait_gather(kk)

            @pl.when(c + 1 <= LC)
            def _():
                issue_idx(c + 1, 1 - kk)
                issue_gather(1 - kk)

            compute(kk)
            base = wid * EPT + c * B
            pltpu.sync_copy(rlo, relout.at[pl.ds(base, B)])
            scatter_sync(kk)

        return carry

    lax.fori_loop(0, NCH // 2, outer_body, 0)
    plsc.subcore_barrier()

    @pl.when(cid == 0)
    def _():
        pltpu.sync_copy(acc.at[pl.ds(r0, RPT)], out0.at[pl.ds(r0, RPT)])

    @pl.when(cid == 1)
    def _():
        pltpu.sync_copy(acc.at[pl.ds(r0, RPT)], out1.at[pl.ds(r0, RPT)])


_kenn_sc = functools.partial(
    pl.kernel,
    out_type=[
        jax.ShapeDtypeStruct((N_PAD, ZC), jnp.float32),
        jax.ShapeDtypeStruct((N_PAD, ZC), jnp.float32),
        jax.ShapeDtypeStruct((E_PAD,), jnp.float32),
    ],
    mesh=plsc.VectorSubcoreMesh(core_axis_name="c", subcore_axis_name="s"),
    compiler_params=pltpu.CompilerParams(
        needs_layout_passes=False, use_tc_tiling_on_sc=False),
    scratch_types=(
        [pltpu.VMEM_SHARED((N_PAD, ZC), jnp.float32)]
        + [pltpu.VMEM((K, 128), jnp.int32)] * 4
        + [pltpu.VMEM((B,), jnp.float32)] * 2
        + [pltpu.VMEM((B, ZC), jnp.float32)] * 6
        + [pltpu.VMEM((B,), jnp.float32)]
        + [pltpu.VMEM((C * 16,), jnp.float32)]
        + [pltpu.SemaphoreType.DMA] * 3
    ),
)(_kenn_body)


def kernel(features, relations, sx, sy, W1, b1, W2, b2, clause_w):
    f32 = jnp.float32
    fpad = jnp.concatenate(
        [features, jnp.zeros((N_PAD - N, D), f32)], axis=0)
    w2p = jnp.concatenate([W2, jnp.zeros((HID, ZC - 8), f32)], axis=1)
    b2p = jnp.concatenate([b2, jnp.zeros((ZC - 8,), f32)]).reshape(1, ZC)
    zt = _mlp(fpad, W1, b1.reshape(1, HID), w2p, b2p)

    pad_e = E_PAD - E
    sxp = jnp.concatenate(
        [sx, jnp.full((pad_e,), JUNK_ROW, jnp.int32)]).reshape(ER, 128)
    syp = jnp.concatenate(
        [sy, jnp.full((pad_e,), JUNK_ROW, jnp.int32)]).reshape(ER, 128)
    relc = jnp.concatenate([relations.reshape(-1), jnp.zeros((pad_e,), f32)])
    wspb = jnp.broadcast_to(
        jax.nn.softplus(clause_w)[:, :, None], (3, C, 16)).reshape(3, C * 16)
    ztab = jnp.zeros((N_PAD, ZC), f32)

    out0 = out1 = None
    for l in range(3):
        out0, out1, relc = _kenn_sc(zt, sxp, syp, relc, ztab, wspb[l])
        if l < 2:
            zt = _combine(zt, out0, out1)

    packed = _epi(zt, out0, out1)
    return (packed[:, 0:4], packed[:, 4:8], packed[:, 8:12], packed[:, 12:16])
